# baseline (speedup 1.0000x reference)
import functools

import jax
import jax.numpy as jnp
import numpy as np
from jax import lax
from jax.experimental import pallas as pl
from jax.experimental.pallas import tpu as pltpu
from jax.experimental.pallas import tpu_sc as plsc

FEAT = 256
P = 4
NOFF = 8
K = P * NOFF
N_ACT = 10000
N_PAS = 30000
N_PAD = 10240

_OFFS8 = np.array([[-1, -1], [0, -1], [1, -1], [-1, 0],
                   [1, 0], [-1, 1], [0, 1], [1, 1]], dtype=np.int32)
_DX = np.array([(p + 1) * _OFFS8[o, 0] for p in range(P) for o in range(NOFF)],
               dtype=np.int32).reshape(1, K)
_DY = np.array([(p + 1) * _OFFS8[o, 1] for p in range(P) for o in range(NOFF)],
               dtype=np.int32).reshape(1, K)
_GSUM = np.zeros((K, K), dtype=np.float32)
for _j in range(K):
    for _i in range(K):
        if _i // P == _j // P:
            _GSUM[_i, _j] = 1.0
_PERM = np.zeros((K, K), dtype=np.float32)
for _o in range(NOFF):
    for _p in range(P):
        _PERM[_o * P + _p, _p * NOFF + _o] = 1.0


def _matmul_bias_kernel(x_ref, w_ref, b_ref, o_ref):
    o_ref[...] = (jnp.dot(x_ref[...], w_ref[...].T,
                          preferred_element_type=jnp.float32) + b_ref[...])


def _proj(x, w, b, blk):
    n = x.shape[0]
    assert n % blk == 0
    return pl.pallas_call(
        _matmul_bias_kernel,
        grid=(n // blk,),
        in_specs=[
            pl.BlockSpec((blk, FEAT), lambda i: (i, 0)),
            pl.BlockSpec((FEAT, FEAT), lambda i: (0, 0)),
            pl.BlockSpec((1, FEAT), lambda i: (0, 0)),
        ],
        out_specs=pl.BlockSpec((blk, FEAT), lambda i: (i, 0)),
        out_shape=jax.ShapeDtypeStruct((n, FEAT), jnp.float32),
    )(x, w, b.reshape(1, FEAT))


def _wpos_kernel(act_ref, xy_ref, bid_ref, mid_ref, aw_ref, ab_ref,
                 dx_ref, dy_ref, gs_ref, pm_ref, w_out_ref, pos_out_ref,
                 *, blk):
    i = pl.program_id(0)
    aw = (jnp.dot(act_ref[...], aw_ref[...].T,
                  preferred_element_type=jnp.float32) + ab_ref[...])
    m = jnp.max(aw, axis=1, keepdims=True)
    e = jnp.exp(aw - m)
    s = jnp.dot(e, gs_ref[...], preferred_element_type=jnp.float32)
    w = jnp.dot(e / s, pm_ref[...], preferred_element_type=jnp.float32)
    row = i * blk + lax.broadcasted_iota(jnp.int32, (blk, 1), 0)
    w_out_ref[...] = jnp.where(row < N_ACT, w, 0.0)

    x = xy_ref[:, 0:1] + dx_ref[...]
    y = xy_ref[:, 1:2] + dy_ref[...]
    b = bid_ref[...]
    pos0 = b * 16384 + jnp.clip(y, 0, 127) * 128 + jnp.clip(x, 0, 127)
    pos1 = 65536 + b * 4096 + jnp.clip(y, 0, 63) * 64 + jnp.clip(x, 0, 63)
    pos_out_ref[...] = jnp.where(mid_ref[...] == 0, pos0, pos1)


def _weights_and_pos(act_pad, xy_pad, bid_pad, mid_pad, attn_w, attn_b):
    blk = 2048
    grid = N_PAD // blk
    return pl.pallas_call(
        functools.partial(_wpos_kernel, blk=blk),
        grid=(grid,),
        in_specs=[
            pl.BlockSpec((blk, FEAT), lambda i: (i, 0)),
            pl.BlockSpec((blk, 2), lambda i: (i, 0)),
            pl.BlockSpec((blk, 1), lambda i: (i, 0)),
            pl.BlockSpec((blk, 1), lambda i: (i, 0)),
            pl.BlockSpec((K, FEAT), lambda i: (0, 0)),
            pl.BlockSpec((1, K), lambda i: (0, 0)),
            pl.BlockSpec((1, K), lambda i: (0, 0)),
            pl.BlockSpec((1, K), lambda i: (0, 0)),
            pl.BlockSpec((K, K), lambda i: (0, 0)),
            pl.BlockSpec((K, K), lambda i: (0, 0)),
        ],
        out_specs=[
            pl.BlockSpec((blk, K), lambda i: (i, 0)),
            pl.BlockSpec((blk, K), lambda i: (i, 0)),
        ],
        out_shape=[
            jax.ShapeDtypeStruct((N_PAD, K), jnp.float32),
            jax.ShapeDtypeStruct((N_PAD, K), jnp.int32),
        ],
    )(act_pad, xy_pad, bid_pad, mid_pad, attn_w, attn_b.reshape(1, K),
      jnp.asarray(_DX), jnp.asarray(_DY), jnp.asarray(_GSUM),
      jnp.asarray(_PERM))


NC = 2
NS = 16
NW = NC * NS
NB = N_PAD // NW
MAPLEN = 4 * 128 * 128 + 4 * 64 * 64
G = 4
NGRP = NB // G

_sc_mesh = plsc.VectorSubcoreMesh(core_axis_name="c", subcore_axis_name="s")


@functools.partial(
    pl.kernel, mesh=_sc_mesh,
    out_type=jax.ShapeDtypeStruct((N_PAD, FEAT), jnp.float32),
    scratch_types=[
        pltpu.VMEM((MAPLEN,), jnp.int32),
        pltpu.VMEM((NB * K,), jnp.int32),
        pltpu.VMEM((NB * K,), jnp.float32),
        pltpu.VMEM((2, K, FEAT), jnp.float32),
        pltpu.VMEM((2, 1, FEAT), jnp.float32),
        pltpu.SemaphoreType.DMA,
        pltpu.SemaphoreType.DMA,
        pltpu.SemaphoreType.DMA,
        pltpu.SemaphoreType.DMA,
    ],
    compiler_params=pltpu.CompilerParams(needs_layout_passes=False),
)
def _sc_attend(vals_hbm, w_hbm, pos_hbm, map_hbm, out_hbm, map_v, ids_v, w_v,
               rows_v, out_v, gsem0, gsem1, osem0, osem1):
    wid = lax.axis_index("s") * NC + lax.axis_index("c")
    base = wid * (NB * K)
    row_base = wid * NB
    pltpu.sync_copy(map_hbm, map_v)
    pltpu.sync_copy(pos_hbm.at[pl.ds(base, NB * K)], ids_v)
    pltpu.sync_copy(w_hbm.at[pl.ds(base, NB * K)], w_v)

    def map_body(i, carry):
        idx = ids_v[pl.ds(i * 16, 16)]
        ids_v[pl.ds(i * 16, 16)] = plsc.load_gather(map_v, [idx])
        return carry

    lax.fori_loop(0, NB * K // 16, map_body, 0)

    def start(g, buf, sem):
        pltpu.async_copy(
            vals_hbm.at[ids_v.at[pl.ds(g * K, K)]],
            rows_v.at[buf], sem)

    def wait_rows(buf, sem):
        pltpu.make_async_copy(
            vals_hbm.at[ids_v.at[pl.ds(0, K)]],
            rows_v.at[buf], sem).wait()

    def out_start(g, ob, sem):
        pltpu.async_copy(out_v.at[ob],
                         out_hbm.at[pl.ds(row_base + g, 1)], sem)

    def out_wait(ob, sem):
        pltpu.make_async_copy(
            out_v.at[ob], out_hbm.at[pl.ds(0, 1)], sem).wait()

    def compute(g, buf, ob):
        acc = [jnp.zeros((16,), jnp.float32)] * 16
        for k in range(K):
            widx = jnp.zeros((16,), jnp.int32) + (g * K + k)
            wk = plsc.load_gather(w_v, [widx])
            for c in range(16):
                acc[c] = acc[c] + wk * rows_v[buf, k, pl.ds(c * 16, 16)]
        for c in range(16):
            out_v[ob, 0, pl.ds(c * 16, 16)] = acc[c]

    start(0, 0, gsem0)

    def pair_body(i, carry):
        g = i * 2
        start(g + 1, 1, gsem1)
        wait_rows(0, gsem0)

        @pl.when(i >= 1)
        def _():
            out_wait(0, osem0)

        compute(g, 0, 0)
        out_start(g, 0, osem0)

        @pl.when(g + 2 < NB)
        def _():
            start(g + 2, 0, gsem0)

        wait_rows(1, gsem1)

        @pl.when(i >= 1)
        def _():
            out_wait(1, osem1)

        compute(g + 1, 1, 1)
        out_start(g + 1, 1, osem1)
        return carry

    lax.fori_loop(0, NB // 2, pair_body, 0)
    out_wait(0, osem0)
    out_wait(1, osem1)


def kernel(in_act_feats, act_batch_ids, act_map_ids, act_xy_ids, pas_feats,
           id_map0, id_map1, attn_w, attn_b, val_w, val_b, out_w, out_b):
    pad = N_PAD - N_ACT
    act_pad = jnp.pad(in_act_feats, ((0, pad), (0, 0)))
    xy_pad = jnp.pad(act_xy_ids.astype(jnp.int32), ((0, pad), (0, 0)))
    bid_pad = jnp.pad(act_batch_ids.astype(jnp.int32), (0, pad)).reshape(N_PAD, 1)
    mid_pad = jnp.pad(act_map_ids.astype(jnp.int32), (0, pad)).reshape(N_PAD, 1)

    feats = jnp.concatenate([in_act_feats, pas_feats], axis=0)
    vals = _proj(feats, val_w, val_b, blk=2000)

    weights, pos = _weights_and_pos(act_pad, xy_pad, bid_pad, mid_pad,
                                    attn_w, attn_b)

    map_flat = jnp.concatenate([id_map0.reshape(-1), id_map1.reshape(-1)])
    val_feats = _sc_attend(vals, weights.reshape(-1), pos.reshape(-1), map_flat)

    out = _proj(val_feats, out_w, out_b, blk=2048)
    return out[:N_ACT]


# speedup vs baseline: 1.1572x; 1.1572x over previous
import functools

import jax
import jax.numpy as jnp
import numpy as np
from jax import lax
from jax.experimental import pallas as pl
from jax.experimental.pallas import tpu as pltpu
from jax.experimental.pallas import tpu_sc as plsc

FEAT = 256
P = 4
NOFF = 8
K = P * NOFF
N_ACT = 10000
N_PAS = 30000
N_PAD = 10240

_OFFS8 = np.array([[-1, -1], [0, -1], [1, -1], [-1, 0],
                   [1, 0], [-1, 1], [0, 1], [1, 1]], dtype=np.int32)
_DX = np.array([(p + 1) * _OFFS8[o, 0] for p in range(P) for o in range(NOFF)],
               dtype=np.int32).reshape(1, K)
_DY = np.array([(p + 1) * _OFFS8[o, 1] for p in range(P) for o in range(NOFF)],
               dtype=np.int32).reshape(1, K)
_GSUM = np.zeros((K, K), dtype=np.float32)
for _j in range(K):
    for _i in range(K):
        if _i // P == _j // P:
            _GSUM[_i, _j] = 1.0
_PERM = np.zeros((K, K), dtype=np.float32)
for _o in range(NOFF):
    for _p in range(P):
        _PERM[_o * P + _p, _p * NOFF + _o] = 1.0


def _matmul_bias_kernel(x_ref, w_ref, b_ref, o_ref):
    o_ref[...] = (jnp.dot(x_ref[...], w_ref[...].T,
                          preferred_element_type=jnp.float32) + b_ref[...])


def _proj(x, w, b, blk):
    n = x.shape[0]
    assert n % blk == 0
    return pl.pallas_call(
        _matmul_bias_kernel,
        grid=(n // blk,),
        in_specs=[
            pl.BlockSpec((blk, FEAT), lambda i: (i, 0)),
            pl.BlockSpec((FEAT, FEAT), lambda i: (0, 0)),
            pl.BlockSpec((1, FEAT), lambda i: (0, 0)),
        ],
        out_specs=pl.BlockSpec((blk, FEAT), lambda i: (i, 0)),
        out_shape=jax.ShapeDtypeStruct((n, FEAT), jnp.float32),
    )(x, w, b.reshape(1, FEAT))


def _wpos_kernel(act_ref, xy_ref, bid_ref, mid_ref, aw_ref, ab_ref,
                 dx_ref, dy_ref, gs_ref, pm_ref, w_out_ref, pos_out_ref,
                 *, blk):
    i = pl.program_id(0)
    aw = (jnp.dot(act_ref[...], aw_ref[...].T,
                  preferred_element_type=jnp.float32) + ab_ref[...])
    m = jnp.max(aw, axis=1, keepdims=True)
    e = jnp.exp(aw - m)
    s = jnp.dot(e, gs_ref[...], preferred_element_type=jnp.float32)
    w = jnp.dot(e / s, pm_ref[...], preferred_element_type=jnp.float32)
    row = i * blk + lax.broadcasted_iota(jnp.int32, (blk, 1), 0)
    w_out_ref[...] = jnp.where(row < N_ACT, w, 0.0)

    x = xy_ref[:, 0:1] + dx_ref[...]
    y = xy_ref[:, 1:2] + dy_ref[...]
    b = bid_ref[...]
    pos0 = b * 16384 + jnp.clip(y, 0, 127) * 128 + jnp.clip(x, 0, 127)
    pos1 = 65536 + b * 4096 + jnp.clip(y, 0, 63) * 64 + jnp.clip(x, 0, 63)
    pos_out_ref[...] = jnp.where(mid_ref[...] == 0, pos0, pos1)


def _weights_and_pos(act_pad, xy_pad, bid_pad, mid_pad, attn_w, attn_b):
    blk = 2048
    grid = N_PAD // blk
    return pl.pallas_call(
        functools.partial(_wpos_kernel, blk=blk),
        grid=(grid,),
        in_specs=[
            pl.BlockSpec((blk, FEAT), lambda i: (i, 0)),
            pl.BlockSpec((blk, 2), lambda i: (i, 0)),
            pl.BlockSpec((blk, 1), lambda i: (i, 0)),
            pl.BlockSpec((blk, 1), lambda i: (i, 0)),
            pl.BlockSpec((K, FEAT), lambda i: (0, 0)),
            pl.BlockSpec((1, K), lambda i: (0, 0)),
            pl.BlockSpec((1, K), lambda i: (0, 0)),
            pl.BlockSpec((1, K), lambda i: (0, 0)),
            pl.BlockSpec((K, K), lambda i: (0, 0)),
            pl.BlockSpec((K, K), lambda i: (0, 0)),
        ],
        out_specs=[
            pl.BlockSpec((blk, K), lambda i: (i, 0)),
            pl.BlockSpec((blk, K), lambda i: (i, 0)),
        ],
        out_shape=[
            jax.ShapeDtypeStruct((N_PAD, K), jnp.float32),
            jax.ShapeDtypeStruct((N_PAD, K), jnp.int32),
        ],
    )(act_pad, xy_pad, bid_pad, mid_pad, attn_w, attn_b.reshape(1, K),
      jnp.asarray(_DX), jnp.asarray(_DY), jnp.asarray(_GSUM),
      jnp.asarray(_PERM))


NC = 2
NS = 16
NW = NC * NS
NB = N_PAD // NW
MAPLEN = 4 * 128 * 128 + 4 * 64 * 64
G = 4
NGRP = NB // G

_sc_mesh = plsc.VectorSubcoreMesh(core_axis_name="c", subcore_axis_name="s")


@functools.partial(
    pl.kernel, mesh=_sc_mesh,
    out_type=jax.ShapeDtypeStruct((N_PAD * K,), jnp.int32),
    scratch_types=[
        pltpu.VMEM((MAPLEN,), jnp.int32),
        pltpu.VMEM((NB * K,), jnp.int32),
        pltpu.VMEM((NB * K,), jnp.int32),
    ],
    compiler_params=pltpu.CompilerParams(needs_layout_passes=False),
)
def _sc_map_gather(map_hbm, pos_hbm, out_hbm, map_v, pos_v, ids_v):
    wid = lax.axis_index("s") * NC + lax.axis_index("c")
    base = wid * (NB * K)
    pltpu.sync_copy(map_hbm, map_v)
    pltpu.sync_copy(pos_hbm.at[pl.ds(base, NB * K)], pos_v)

    def body(i, carry):
        idx = pos_v[pl.ds(i * 16, 16)]
        ids_v[pl.ds(i * 16, 16)] = plsc.load_gather(map_v, [idx])
        return carry

    lax.fori_loop(0, NB * K // 16, body, 0)
    pltpu.sync_copy(ids_v, out_hbm.at[pl.ds(base, NB * K)])


@functools.partial(
    pl.kernel, mesh=_sc_mesh,
    out_type=jax.ShapeDtypeStruct((N_PAD, FEAT), jnp.float32),
    scratch_types=[
        pltpu.VMEM((NB * K,), jnp.int32),
        pltpu.VMEM((NB * K,), jnp.float32),
        pltpu.VMEM((2, G * K, FEAT), jnp.float32),
        pltpu.VMEM((2, G, FEAT), jnp.float32),
        pltpu.SemaphoreType.DMA,
        pltpu.SemaphoreType.DMA,
        pltpu.SemaphoreType.DMA,
        pltpu.SemaphoreType.DMA,
    ],
    compiler_params=pltpu.CompilerParams(needs_layout_passes=False),
)
def _sc_wsum(vals_hbm, w_hbm, ids_hbm, out_hbm, ids_v, w_v, rows_v, out_v,
             gsem0, gsem1, osem0, osem1):
    wid = lax.axis_index("s") * NC + lax.axis_index("c")
    base = wid * (NB * K)
    row_base = wid * NB
    pltpu.sync_copy(ids_hbm.at[pl.ds(base, NB * K)], ids_v)
    pltpu.sync_copy(w_hbm.at[pl.ds(base, NB * K)], w_v)

    def start(g, buf, sem):
        pltpu.async_copy(
            vals_hbm.at[ids_v.at[pl.ds(g * (G * K), G * K)]],
            rows_v.at[buf], sem)

    def wait_rows(buf, sem):
        pltpu.make_async_copy(
            vals_hbm.at[ids_v.at[pl.ds(0, G * K)]],
            rows_v.at[buf], sem).wait()

    def out_start(g, ob, sem):
        pltpu.async_copy(out_v.at[ob],
                         out_hbm.at[pl.ds(row_base + g * G, G)], sem)

    def out_wait(ob, sem):
        pltpu.make_async_copy(
            out_v.at[ob], out_hbm.at[pl.ds(0, G)], sem).wait()

    def compute(g, buf, ob):
        def act_body(a, carry):
            wbase = (g * G + a) * K
            acc = [jnp.zeros((16,), jnp.float32)] * 16
            for k in range(K):
                widx = jnp.zeros((16,), jnp.int32) + (wbase + k)
                wk = plsc.load_gather(w_v, [widx])
                r = a * K + k
                for c in range(16):
                    acc[c] = acc[c] + wk * rows_v[buf, r, pl.ds(c * 16, 16)]
            for c in range(16):
                out_v[ob, a, pl.ds(c * 16, 16)] = acc[c]
            return carry

        lax.fori_loop(0, G, act_body, 0)

    start(0, 0, gsem0)

    def pair_body(i, carry):
        g = i * 2
        start(g + 1, 1, gsem1)
        wait_rows(0, gsem0)

        @pl.when(i >= 1)
        def _():
            out_wait(0, osem0)

        compute(g, 0, 0)
        out_start(g, 0, osem0)

        @pl.when(g + 2 < NGRP)
        def _():
            start(g + 2, 0, gsem0)

        wait_rows(1, gsem1)

        @pl.when(i >= 1)
        def _():
            out_wait(1, osem1)

        compute(g + 1, 1, 1)
        out_start(g + 1, 1, osem1)
        return carry

    lax.fori_loop(0, NGRP // 2, pair_body, 0)
    out_wait(0, osem0)
    out_wait(1, osem1)


def kernel(in_act_feats, act_batch_ids, act_map_ids, act_xy_ids, pas_feats,
           id_map0, id_map1, attn_w, attn_b, val_w, val_b, out_w, out_b):
    pad = N_PAD - N_ACT
    act_pad = jnp.pad(in_act_feats, ((0, pad), (0, 0)))
    xy_pad = jnp.pad(act_xy_ids.astype(jnp.int32), ((0, pad), (0, 0)))
    bid_pad = jnp.pad(act_batch_ids.astype(jnp.int32), (0, pad)).reshape(N_PAD, 1)
    mid_pad = jnp.pad(act_map_ids.astype(jnp.int32), (0, pad)).reshape(N_PAD, 1)

    feats = jnp.concatenate([in_act_feats, pas_feats], axis=0)
    vals = _proj(feats, val_w, val_b, blk=2000)

    weights, pos = _weights_and_pos(act_pad, xy_pad, bid_pad, mid_pad,
                                    attn_w, attn_b)

    map_flat = jnp.concatenate([id_map0.reshape(-1), id_map1.reshape(-1)])
    feat_ids_flat = _sc_map_gather(map_flat, pos.reshape(-1))
    val_feats = _sc_wsum(vals, weights.reshape(-1), feat_ids_flat)

    out = _proj(val_feats, out_w, out_b, blk=2048)
    return out[:N_ACT]


# speedup vs baseline: 1.1626x; 1.0046x over previous
import functools

import jax
import jax.numpy as jnp
import numpy as np
from jax import lax
from jax.experimental import pallas as pl
from jax.experimental.pallas import tpu as pltpu
from jax.experimental.pallas import tpu_sc as plsc

FEAT = 256
P = 4
NOFF = 8
K = P * NOFF
N_ACT = 10000
N_PAS = 30000
N_PAD = 10240

_OFFS8 = np.array([[-1, -1], [0, -1], [1, -1], [-1, 0],
                   [1, 0], [-1, 1], [0, 1], [1, 1]], dtype=np.int32)
_DX = np.array([(p + 1) * _OFFS8[o, 0] for p in range(P) for o in range(NOFF)],
               dtype=np.int32).reshape(1, K)
_DY = np.array([(p + 1) * _OFFS8[o, 1] for p in range(P) for o in range(NOFF)],
               dtype=np.int32).reshape(1, K)
_GSUM = np.zeros((K, K), dtype=np.float32)
for _j in range(K):
    for _i in range(K):
        if _i // P == _j // P:
            _GSUM[_i, _j] = 1.0
_PERM = np.zeros((K, K), dtype=np.float32)
for _o in range(NOFF):
    for _p in range(P):
        _PERM[_o * P + _p, _p * NOFF + _o] = 1.0


def _matmul_bias_kernel(x_ref, w_ref, b_ref, o_ref):
    o_ref[...] = (jnp.dot(x_ref[...], w_ref[...].T,
                          preferred_element_type=jnp.float32) + b_ref[...])


def _proj(x, w, b, blk):
    n = x.shape[0]
    assert n % blk == 0
    return pl.pallas_call(
        _matmul_bias_kernel,
        grid=(n // blk,),
        in_specs=[
            pl.BlockSpec((blk, FEAT), lambda i: (i, 0)),
            pl.BlockSpec((FEAT, FEAT), lambda i: (0, 0)),
            pl.BlockSpec((1, FEAT), lambda i: (0, 0)),
        ],
        out_specs=pl.BlockSpec((blk, FEAT), lambda i: (i, 0)),
        out_shape=jax.ShapeDtypeStruct((n, FEAT), jnp.float32),
    )(x, w, b.reshape(1, FEAT))


def _wpos_kernel(act_ref, xy_ref, bid_ref, mid_ref, aw_ref, ab_ref,
                 dx_ref, dy_ref, gs_ref, pm_ref, w_out_ref, pos_out_ref,
                 *, blk):
    i = pl.program_id(0)
    aw = (jnp.dot(act_ref[...], aw_ref[...].T,
                  preferred_element_type=jnp.float32) + ab_ref[...])
    m = jnp.max(aw, axis=1, keepdims=True)
    e = jnp.exp(aw - m)
    s = jnp.dot(e, gs_ref[...], preferred_element_type=jnp.float32)
    w = jnp.dot(e / s, pm_ref[...], preferred_element_type=jnp.float32)
    row = i * blk + lax.broadcasted_iota(jnp.int32, (blk, 1), 0)
    w_out_ref[...] = jnp.where(row < N_ACT, w, 0.0)

    x = xy_ref[:, 0:1] + dx_ref[...]
    y = xy_ref[:, 1:2] + dy_ref[...]
    b = bid_ref[...]
    pos0 = b * 16384 + jnp.clip(y, 0, 127) * 128 + jnp.clip(x, 0, 127)
    pos1 = 65536 + b * 4096 + jnp.clip(y, 0, 63) * 64 + jnp.clip(x, 0, 63)
    pos_out_ref[...] = jnp.where(mid_ref[...] == 0, pos0, pos1)


def _weights_and_pos(act_pad, xy_pad, bid_pad, mid_pad, attn_w, attn_b):
    blk = 2048
    grid = N_PAD // blk
    return pl.pallas_call(
        functools.partial(_wpos_kernel, blk=blk),
        grid=(grid,),
        in_specs=[
            pl.BlockSpec((blk, FEAT), lambda i: (i, 0)),
            pl.BlockSpec((blk, 2), lambda i: (i, 0)),
            pl.BlockSpec((blk, 1), lambda i: (i, 0)),
            pl.BlockSpec((blk, 1), lambda i: (i, 0)),
            pl.BlockSpec((K, FEAT), lambda i: (0, 0)),
            pl.BlockSpec((1, K), lambda i: (0, 0)),
            pl.BlockSpec((1, K), lambda i: (0, 0)),
            pl.BlockSpec((1, K), lambda i: (0, 0)),
            pl.BlockSpec((K, K), lambda i: (0, 0)),
            pl.BlockSpec((K, K), lambda i: (0, 0)),
        ],
        out_specs=[
            pl.BlockSpec((blk, K), lambda i: (i, 0)),
            pl.BlockSpec((blk, K), lambda i: (i, 0)),
        ],
        out_shape=[
            jax.ShapeDtypeStruct((N_PAD, K), jnp.float32),
            jax.ShapeDtypeStruct((N_PAD, K), jnp.int32),
        ],
    )(act_pad, xy_pad, bid_pad, mid_pad, attn_w, attn_b.reshape(1, K),
      jnp.asarray(_DX), jnp.asarray(_DY), jnp.asarray(_GSUM),
      jnp.asarray(_PERM))


NC = 2
NS = 16
NW = NC * NS
NB = N_PAD // NW
MAPLEN = 4 * 128 * 128 + 4 * 64 * 64
G = 4
NGRP = NB // G

_sc_mesh = plsc.VectorSubcoreMesh(core_axis_name="c", subcore_axis_name="s")


@functools.partial(
    pl.kernel, mesh=_sc_mesh,
    out_type=jax.ShapeDtypeStruct((N_PAD * K,), jnp.int32),
    scratch_types=[
        pltpu.VMEM((MAPLEN,), jnp.int32),
        pltpu.VMEM((NB * K,), jnp.int32),
        pltpu.VMEM((NB * K,), jnp.int32),
    ],
    compiler_params=pltpu.CompilerParams(needs_layout_passes=False),
)
def _sc_map_gather(map_hbm, pos_hbm, out_hbm, map_v, pos_v, ids_v):
    wid = lax.axis_index("s") * NC + lax.axis_index("c")
    base = wid * (NB * K)
    pltpu.sync_copy(map_hbm, map_v)
    pltpu.sync_copy(pos_hbm.at[pl.ds(base, NB * K)], pos_v)

    def body(i, carry):
        idx = pos_v[pl.ds(i * 16, 16)]
        ids_v[pl.ds(i * 16, 16)] = plsc.load_gather(map_v, [idx])
        return carry

    lax.fori_loop(0, NB * K // 16, body, 0)
    pltpu.sync_copy(ids_v, out_hbm.at[pl.ds(base, NB * K)])


@functools.partial(
    pl.kernel, mesh=_sc_mesh,
    out_type=jax.ShapeDtypeStruct((N_PAD, FEAT), jnp.float32),
    scratch_types=[
        pltpu.VMEM((NB * K,), jnp.int32),
        pltpu.VMEM((NB * K,), jnp.float32),
        pltpu.VMEM((2, G * K, FEAT), jnp.float32),
        pltpu.VMEM((2, G, FEAT), jnp.float32),
        pltpu.SemaphoreType.DMA,
        pltpu.SemaphoreType.DMA,
        pltpu.SemaphoreType.DMA,
        pltpu.SemaphoreType.DMA,
    ],
    compiler_params=pltpu.CompilerParams(needs_layout_passes=False),
)
def _sc_wsum(vals_hbm, w_hbm, ids_hbm, out_hbm, ids_v, w_v, rows_v, out_v,
             gsem0, gsem1, osem0, osem1):
    wid = lax.axis_index("s") * NC + lax.axis_index("c")
    base = wid * (NB * K)
    row_base = wid * NB
    pltpu.sync_copy(ids_hbm.at[pl.ds(base, NB * K)], ids_v)
    pltpu.sync_copy(w_hbm.at[pl.ds(base, NB * K)], w_v)

    def start(g, buf, sem):
        pltpu.async_copy(
            vals_hbm.at[ids_v.at[pl.ds(g * (G * K), G * K)]],
            rows_v.at[buf], sem)

    def wait_rows(buf, sem):
        pltpu.make_async_copy(
            vals_hbm.at[ids_v.at[pl.ds(0, G * K)]],
            rows_v.at[buf], sem).wait()

    def out_start(g, ob, sem):
        pltpu.async_copy(out_v.at[ob],
                         out_hbm.at[pl.ds(row_base + g * G, G)], sem)

    def out_wait(ob, sem):
        pltpu.make_async_copy(
            out_v.at[ob], out_hbm.at[pl.ds(0, G)], sem).wait()

    def compute(g, buf, ob):
        def act_body(a, carry):
            wbase = (g * G + a) * K
            acc = [jnp.zeros((16,), jnp.float32)] * 16
            for k in range(K):
                widx = jnp.zeros((16,), jnp.int32) + (wbase + k)
                wk = plsc.load_gather(w_v, [widx])
                r = a * K + k
                for c in range(16):
                    acc[c] = acc[c] + wk * rows_v[buf, r, pl.ds(c * 16, 16)]
            for c in range(16):
                out_v[ob, a, pl.ds(c * 16, 16)] = acc[c]
            return carry

        lax.fori_loop(0, G, act_body, 0)

    start(0, 0, gsem0)

    def pair_body(i, carry):
        g = i * 2
        start(g + 1, 1, gsem1)
        wait_rows(0, gsem0)

        @pl.when(i >= 1)
        def _():
            out_wait(0, osem0)

        compute(g, 0, 0)
        out_start(g, 0, osem0)

        @pl.when(g + 2 < NGRP)
        def _():
            start(g + 2, 0, gsem0)

        wait_rows(1, gsem1)

        @pl.when(i >= 1)
        def _():
            out_wait(1, osem1)

        compute(g + 1, 1, 1)
        out_start(g + 1, 1, osem1)
        return carry

    lax.fori_loop(0, NGRP // 2, pair_body, 0)
    out_wait(0, osem0)
    out_wait(1, osem1)


def kernel(in_act_feats, act_batch_ids, act_map_ids, act_xy_ids, pas_feats,
           id_map0, id_map1, attn_w, attn_b, val_w, val_b, out_w, out_b):
    pad = N_PAD - N_ACT
    act_pad = jnp.pad(in_act_feats, ((0, pad), (0, 0)))
    xy_pad = jnp.pad(act_xy_ids.astype(jnp.int32), ((0, pad), (0, 0)))
    bid_pad = jnp.pad(act_batch_ids.astype(jnp.int32), (0, pad)).reshape(N_PAD, 1)
    mid_pad = jnp.pad(act_map_ids.astype(jnp.int32), (0, pad)).reshape(N_PAD, 1)

    feats = jnp.concatenate([in_act_feats, pas_feats], axis=0)
    vals = _proj(feats, val_w, val_b, blk=4000)

    weights, pos = _weights_and_pos(act_pad, xy_pad, bid_pad, mid_pad,
                                    attn_w, attn_b)

    map_flat = jnp.concatenate([id_map0.reshape(-1), id_map1.reshape(-1)])
    feat_ids_flat = _sc_map_gather(map_flat, pos.reshape(-1))
    val_feats = _sc_wsum(vals, weights.reshape(-1), feat_ids_flat)

    out = _proj(val_feats, out_w, out_b, blk=5120)
    return out[:N_ACT]
